# baseline (speedup 1.0000x reference)
import functools

import jax
import jax.numpy as jnp
from jax import lax
from jax.experimental import pallas as pl
from jax.experimental.pallas import tpu as pltpu
from jax.experimental.pallas import tpu_sc as plsc

DIM = 64
NUM_WORKERS = 32
CHUNK = 128
GROUP = 1


def _gather_body(table_hbm, idx_hbm, out_hbm, idx_v, rows_a, rows_b, *sems):
    n_per_w = idx_hbm.shape[1]
    n_group = n_per_w // (GROUP * CHUNK)
    rows = (rows_a, rows_b)
    gsem = sems[0:2]
    wsem = sems[2:4]
    g_rows = GROUP * CHUNK

    wid = lax.axis_index("s") * 2 + lax.axis_index("c")
    base = wid * n_per_w
    pltpu.sync_copy(idx_hbm.at[wid], idx_v)

    def gather(g, p):
        src = table_hbm.at[idx_v.at[pl.ds(g * g_rows, g_rows)]]
        return pltpu.make_async_copy(src, rows[p], gsem[p])

    def write(g, p):
        dst = out_hbm.at[pl.ds(base + g * g_rows, g_rows)]
        return pltpu.make_async_copy(rows[p], dst, wsem[p])

    gather(0, 0).start()
    gather(1, 1).start()
    for g in range(n_group):
        p = g % 2
        gather(g, p).wait()
        write(g, p).start()
        if g >= 2:
            write(0, p).wait()
        if g + 2 < n_group:
            gather(g + 2, p).start()
    write(0, 0).wait()
    write(0, 1).wait()


@jax.jit
def _run(idx, table):
    n_total = idx.shape[0] * idx.shape[1]
    mesh = plsc.VectorSubcoreMesh(core_axis_name="c", subcore_axis_name="s")
    k = functools.partial(
        pl.kernel,
        mesh=mesh,
        compiler_params=pltpu.CompilerParams(use_tc_tiling_on_sc=False),
        out_type=jax.ShapeDtypeStruct((n_total, DIM), jnp.float32),
        scratch_types=[pltpu.VMEM((idx.shape[1],), jnp.int32)]
        + [pltpu.VMEM((GROUP * CHUNK, DIM), jnp.float32) for _ in range(2)]
        + [pltpu.SemaphoreType.DMA for _ in range(4)],
    )(_gather_body)
    return k(table, idx)


def kernel(input_data, embeddings_matrix):
    b, h = input_data.shape
    n_total = b * h
    assert n_total % (NUM_WORKERS * CHUNK * GROUP) == 0
    idx = input_data.astype(jnp.int32).reshape(NUM_WORKERS, n_total // NUM_WORKERS)
    out = _run(idx, embeddings_matrix)
    return out.reshape(b, h, DIM)


# speedup vs baseline: 1.0080x; 1.0080x over previous
import functools

import jax
import jax.numpy as jnp
from jax import lax
from jax.experimental import pallas as pl
from jax.experimental.pallas import tpu as pltpu
from jax.experimental.pallas import tpu_sc as plsc

DIM = 64
NUM_WORKERS = 32
RGROUP = 8


def _gather_body(table_hbm, idx_hbm, out_hbm, idx_v, rows_a, rows_b, *sems):
    rows_per_w = idx_hbm.shape[0] // NUM_WORKERS
    n_group = rows_per_w // RGROUP
    rows = (rows_a, rows_b)
    gsem = sems[0:2]
    wsem = sems[2:4]

    wid = lax.axis_index("s") * 2 + lax.axis_index("c")
    row0 = wid * rows_per_w
    pltpu.sync_copy(idx_hbm.at[pl.ds(row0, rows_per_w)], idx_v)

    def gathers(g, p, op):
        for i in range(RGROUP):
            r = g * RGROUP + i
            d = pltpu.make_async_copy(table_hbm.at[idx_v.at[r]], rows[p].at[i], gsem[p])
            (d.start if op == "start" else d.wait)()

    def write(g, p):
        dst = out_hbm.at[pl.ds(row0 + g * RGROUP, RGROUP)]
        return pltpu.make_async_copy(rows[p], dst, wsem[p])

    gathers(0, 0, "start")
    gathers(1, 1, "start")
    for g in (0, 1):
        gathers(g, g, "wait")
        write(g, g).start()
        write(0, g).wait()
        gathers(g + 2, g, "start")

    def body(k, carry):
        for p in (0, 1):
            g = 2 * k + p
            gathers(g, p, "wait")
            write(g, p).start()
            write(0, p).wait()
            gathers(g + 2, p, "start")
        return carry

    lax.fori_loop(1, n_group // 2 - 1, body, 0)

    for g in (n_group - 2, n_group - 1):
        p = g % 2
        gathers(g, p, "wait")
        write(g, p).start()
        write(0, p).wait()


@jax.jit
def _run(idx, table):
    b, h = idx.shape
    mesh = plsc.VectorSubcoreMesh(core_axis_name="c", subcore_axis_name="s")
    k = functools.partial(
        pl.kernel,
        mesh=mesh,
        compiler_params=pltpu.CompilerParams(use_tc_tiling_on_sc=False),
        out_type=jax.ShapeDtypeStruct((b, h, DIM), jnp.float32),
        scratch_types=[pltpu.VMEM((b // NUM_WORKERS, h), jnp.int32)]
        + [pltpu.VMEM((RGROUP, h, DIM), jnp.float32) for _ in range(2)]
        + [pltpu.SemaphoreType.DMA for _ in range(4)],
    )(_gather_body)
    return k(table, idx)


def kernel(input_data, embeddings_matrix):
    return _run(input_data.astype(jnp.int32), embeddings_matrix)
